# baseline (speedup 1.0000x reference)
import functools

import jax
import jax.numpy as jnp
from jax import lax
from jax.experimental import pallas as pl
from jax.experimental.pallas import tpu as pltpu
from jax.experimental.pallas import tpu_sc as plsc

NC = 2
NS = 16
NW = NC * NS
D = 32
CHUNK = 256


@functools.partial(jax.jit, static_argnames=("b_total",))
def _embed_lookup(idx_flat, weight128, b_total):
    b_per_w = b_total // NW
    n_chunks = b_per_w // CHUNK
    n_groups = CHUNK // 16
    out_rows_per_chunk = CHUNK // 4
    mesh = plsc.VectorSubcoreMesh(core_axis_name="c", subcore_axis_name="s")

    @functools.partial(
        pl.kernel,
        out_type=jax.ShapeDtypeStruct((b_total // 4, 128), jnp.float32),
        mesh=mesh,
        scratch_types=[
            pltpu.VMEM((b_per_w,), jnp.int32),
            pltpu.VMEM((b_per_w,), jnp.int32),
            pltpu.VMEM((2, CHUNK, 128), jnp.float32),
            pltpu.VMEM((out_rows_per_chunk, 128), jnp.float32),
            pltpu.SemaphoreType.DMA,
            pltpu.SemaphoreType.DMA,
        ],
        compiler_params=pltpu.CompilerParams(needs_layout_passes=False),
    )
    def k(idx_hbm, table_hbm, out_hbm, idx_v, sup_v, rows_v, ext_v,
          gsem0, gsem1):
        wid = lax.axis_index("s") * NC + lax.axis_index("c")
        base = wid * b_per_w
        out_base = wid * (b_per_w // 4)
        pltpu.sync_copy(idx_hbm.at[pl.ds(base, b_per_w)], idx_v)

        lane = lax.iota(jnp.int32, 16)

        def sup_body(g, _):
            ids = idx_v[pl.ds(g * 16, 16)]
            sup_v[pl.ds(g * 16, 16)] = lax.shift_right_logical(ids, 2)
            return 0
        lax.fori_loop(0, b_per_w // 16, sup_body, 0, unroll=8)

        gsems = (gsem0, gsem1)

        def fire(c, buf):
            pltpu.async_copy(
                table_hbm.at[sup_v.at[pl.ds(c * CHUNK, CHUNK)]],
                rows_v.at[buf], gsems[buf])

        def process(c, buf):
            pltpu.make_async_copy(
                table_hbm.at[sup_v.at[pl.ds(0, CHUNK)]],
                rows_v.at[buf], gsems[buf]).wait()
            rows_c = rows_v.at[buf]

            def ext_body(grp, _):
                pos = grp * 16 + lane
                ids = idx_v[pl.ds(c * CHUNK + grp * 16, 16)]
                src_col = (ids & 3) * D
                dst_row = lax.shift_right_logical(pos, 2)
                dst_col = (pos & 3) * D
                for j in range(D):
                    val = plsc.load_gather(rows_c, [pos, src_col + j])
                    plsc.store_scatter(ext_v, [dst_row, dst_col + j], val)
                return 0
            lax.fori_loop(0, n_groups, ext_body, 0)

            pltpu.sync_copy(
                ext_v,
                out_hbm.at[pl.ds(out_base + c * out_rows_per_chunk,
                                 out_rows_per_chunk)])

        fire(0, 0)

        @pl.loop(0, n_chunks, step=2)
        def chunk_pair(i):
            fire(i + 1, 1)
            process(i, 0)

            @pl.when(i + 2 < n_chunks)
            def _():
                fire(i + 2, 0)
            process(i + 1, 1)

    return k(idx_flat, weight128)


def kernel(token_ids, weight):
    b, s = token_ids.shape
    idx_flat = token_ids.reshape(-1).astype(jnp.int32)
    weight128 = weight.reshape(-1, 128)
    out = _embed_lookup(idx_flat, weight128, b * s)
    return out.reshape(b, s, D)
